# baseline (speedup 1.0000x reference)
import dataclasses
import functools

import jax
import jax.numpy as jnp
from jax import lax
from jax.experimental import pallas as pl
from jax.experimental.pallas import tpu as pltpu
from jax.experimental.pallas import tpu_sc as plsc

NC, NS, LANES = 2, 16, 16
NW = NC * NS


def _sc_compiler_params():
    cp = pltpu.CompilerParams(use_tc_tiling_on_sc=False)
    if "needs_layout_passes" in pltpu.CompilerParams.__dataclass_fields__:
        cp = dataclasses.replace(cp, needs_layout_passes=False)
    return cp


def _sc_embed(x_t, tok, pos_et):
    L, B = x_t.shape
    V, E = tok.shape
    BW = B // NW
    NBUF = 3
    NOB = 2

    mesh = plsc.VectorSubcoreMesh(core_axis_name="c", subcore_axis_name="s")

    @functools.partial(
        pl.kernel,
        out_type=jax.ShapeDtypeStruct((L, E, B), jnp.float32),
        mesh=mesh,
        compiler_params=_sc_compiler_params(),
        scratch_types=[
            pltpu.VMEM((L, BW), jnp.int32),
            pltpu.VMEM((NBUF, BW, E), jnp.float32),
            pltpu.VMEM((NOB, E, BW), jnp.float32),
            pltpu.VMEM((E, L), jnp.float32),
            pltpu.SemaphoreType.DMA((NBUF,)),
            pltpu.SemaphoreType.DMA((NOB,)),
        ],
    )
    def k(xt_hbm, tok_hbm, pos_hbm, out_hbm, idxs_v, rows_v, outb_v, pos_v,
          gsem, osem):
        wid = lax.axis_index("s") * NC + lax.axis_index("c")
        bbase = wid * BW

        pltpu.sync_copy(pos_hbm, pos_v)
        pltpu.sync_copy(xt_hbm.at[:, pl.ds(bbase, BW)], idxs_v)

        def start_gather(t, jb):
            pltpu.async_copy(
                tok_hbm.at[idxs_v.at[t]], rows_v.at[jb], gsem.at[jb])

        def wait_gather(t, jb):
            pltpu.make_async_copy(
                tok_hbm.at[idxs_v.at[t]], rows_v.at[jb], gsem.at[jb]).wait()

        def start_out(t, jb, q):
            pltpu.async_copy(
                outb_v.at[q], out_hbm.at[t, :, pl.ds(bbase, BW)], osem.at[q])

        def wait_out(t, q):
            pltpu.make_async_copy(
                outb_v.at[q], out_hbm.at[t, :, pl.ds(bbase, BW)],
                osem.at[q]).wait()

        iota = lax.iota(jnp.int32, LANES)

        def transpose_add(t, jb, q):
            lvec = jnp.full((LANES,), t, jnp.int32)

            @pl.loop(0, E)
            def _(e):
                evec = jnp.full((LANES,), e, jnp.int32)
                pvec = plsc.load_gather(pos_v, [evec, lvec])
                for bb in range(BW // LANES):
                    bvec = iota + (bb * LANES)
                    val = plsc.load_gather(rows_v.at[jb], [bvec, evec])
                    outb_v[q, e, pl.ds(bb * LANES, LANES)] = val + pvec

        for t in range(NBUF):
            start_gather(t, t)

        def step(t, jb, q, prefetch, outwait):
            wait_gather(t, jb)
            if outwait:
                wait_out(t - NOB, q)
            transpose_add(t, jb, q)
            start_out(t, jb, q)
            if prefetch:
                start_gather(t + NBUF, jb)

        for t in range(NOB):
            step(t, t % NBUF, t % NOB, True, False)

        STEP = NBUF * NOB
        body_lo = NOB
        body_hi = L - NBUF
        n_mid = ((body_hi - body_lo) // STEP) * STEP
        mid_hi = body_lo + n_mid

        @pl.loop(body_lo, mid_hi, step=STEP)
        def _(t0):
            for j in range(STEP):
                t = t0 + j
                step(t, (body_lo + j) % NBUF, j % NOB, True, True)

        for t in range(mid_hi, L):
            step(t, t % NBUF, t % NOB, t + NBUF < L, True)

        for t in range(L - NOB, L):
            wait_out(t, t % NOB)

    return k(x_t, tok, pos_et)


@jax.jit
def kernel(x, token_table, pos_table):
    B, L = x.shape
    E = token_table.shape[1]
    x_t = x.T.astype(jnp.int32)
    pos_et = pos_table[:L].T.astype(jnp.float32)
    out_t = _sc_embed(x_t, token_table, pos_et)
    return jnp.transpose(out_t, (2, 0, 1))


# speedup vs baseline: 1.1880x; 1.1880x over previous
import dataclasses
import functools

import jax
import jax.numpy as jnp
from jax import lax
from jax.experimental import pallas as pl
from jax.experimental.pallas import tpu as pltpu
from jax.experimental.pallas import tpu_sc as plsc

NC, NS, LANES = 2, 16, 16
NW = NC * NS
CW = 256


def _sc_compiler_params():
    cp = pltpu.CompilerParams(use_tc_tiling_on_sc=True)
    if "needs_layout_passes" in pltpu.CompilerParams.__dataclass_fields__:
        cp = dataclasses.replace(cp, needs_layout_passes=False)
    return cp


def _sc_pack_table(tok_t, tail2):
    E, V = tok_t.shape
    n_full = V // CW
    per_w = -(-n_full // NW)
    n_tail = tail2.shape[0]

    mesh = plsc.VectorSubcoreMesh(core_axis_name="c", subcore_axis_name="s")

    @functools.partial(
        pl.kernel,
        out_type=jax.ShapeDtypeStruct((V // 2, 2 * E), jnp.float32),
        mesh=mesh,
        compiler_params=_sc_compiler_params(),
        scratch_types=[
            pltpu.VMEM((E, CW + 5), jnp.float32),
            pltpu.VMEM((CW // 2, 2 * E + 1), jnp.float32),
        ],
    )
    def k1(tokt_hbm, tail2_hbm, tab2_hbm, inb_v, outb_v):
        wid = lax.axis_index("s") * NC + lax.axis_index("c")
        iota = lax.iota(jnp.int32, LANES)

        @pl.when(wid == NW - 1)
        def _():
            pltpu.sync_copy(tail2_hbm,
                            tab2_hbm.at[pl.ds(V // 2 - n_tail, n_tail)])

        def do_chunk(cid):
            cs = cid * CW
            pltpu.sync_copy(tokt_hbm.at[:, pl.ds(cs, CW)],
                            inb_v.at[:, pl.ds(0, CW)])

            @plsc.parallel_loop(0, CW, step=1, unroll=2)
            def _(v):
                half = (v & 1) * E
                row = v >> 1
                for c in range(E // LANES):
                    val = plsc.load_gather(
                        inb_v, [iota + c * LANES, jnp.full((LANES,), v,
                                                           jnp.int32)])
                    outb_v[row, pl.ds(half + c * LANES, LANES)] = val

            pltpu.sync_copy(outb_v.at[:, pl.ds(0, 2 * E)],
                            tab2_hbm.at[pl.ds(cid * (CW // 2), CW // 2)])

        @pl.loop(0, per_w)
        def _(i):
            cid = wid + NW * i

            @pl.when(cid < n_full)
            def _():
                do_chunk(cid)

    return k1(tok_t, tail2)


def _sc_embed(x_t, tab2, pos_t):
    L, B = x_t.shape
    E = pos_t.shape[0]
    BW = B // NW
    OW = BW + 1
    NBUF = 3
    NOB = 2
    PL = 256

    mesh = plsc.VectorSubcoreMesh(core_axis_name="c", subcore_axis_name="s")

    @functools.partial(
        pl.kernel,
        out_type=jax.ShapeDtypeStruct((L, E, B), jnp.float32),
        mesh=mesh,
        compiler_params=_sc_compiler_params(),
        scratch_types=[
            pltpu.VMEM((L, BW), jnp.int32),
            pltpu.VMEM((NBUF, BW), jnp.int32),
            pltpu.VMEM((NBUF, BW, 2 * E), jnp.float32),
            pltpu.VMEM((NOB, E, OW), jnp.float32),
            pltpu.VMEM((E, PL), jnp.float32),
            pltpu.SemaphoreType.DMA((NBUF,)),
            pltpu.SemaphoreType.DMA((NOB,)),
        ],
    )
    def k2(xt_hbm, tab2_hbm, pos_hbm, out_hbm, idxs_v, idxp_v, rows_v,
           outb_v, pos_v, gsem, osem):
        wid = lax.axis_index("s") * NC + lax.axis_index("c")
        bbase = wid * BW
        iota = lax.iota(jnp.int32, LANES)

        pltpu.sync_copy(pos_hbm.at[:, pl.ds(0, PL)], pos_v)
        pltpu.sync_copy(xt_hbm.at[:, pl.ds(bbase, BW)], idxs_v)

        def start_gather(t, jb):
            for c in range(BW // LANES):
                sl = pl.ds(c * LANES, LANES)
                idxp_v[jb, sl] = jax.lax.shift_right_logical(
                    idxs_v[t, sl], 1)
            pltpu.async_copy(
                tab2_hbm.at[idxp_v.at[jb]], rows_v.at[jb], gsem.at[jb])

        def wait_gather(t, jb):
            pltpu.make_async_copy(
                tab2_hbm.at[idxp_v.at[jb]], rows_v.at[jb], gsem.at[jb]).wait()

        def start_out(t, q):
            pltpu.async_copy(
                outb_v.at[q, :, pl.ds(0, BW)],
                out_hbm.at[t, :, pl.ds(bbase, BW)], osem.at[q])

        def wait_out(t, q):
            pltpu.make_async_copy(
                outb_v.at[q, :, pl.ds(0, BW)],
                out_hbm.at[t, :, pl.ds(bbase, BW)], osem.at[q]).wait()

        def transpose_add(t, jb, q):
            lvec = jnp.full((LANES,), t, jnp.int32)
            pos_c = [
                plsc.load_gather(pos_v, [iota + c * LANES, lvec])
                for c in range(E // LANES)
            ]

            @plsc.parallel_loop(0, BW, step=1, unroll=2)
            def _(b):
                idxb = plsc.load_gather(
                    idxs_v, [jnp.full((LANES,), t, jnp.int32),
                             jnp.full((LANES,), b, jnp.int32)])
                halfoff = jax.lax.shift_left((idxb & 1), 6)
                bvec = jnp.full((LANES,), b, jnp.int32)
                for c in range(E // LANES):
                    col = halfoff + (c * LANES) + iota
                    val = plsc.load_gather(rows_v.at[jb], [bvec, col])
                    plsc.store_scatter(
                        outb_v.at[q], [iota + c * LANES, bvec],
                        val + pos_c[c])

        for t in range(NBUF):
            start_gather(t, t)

        def step(t, jb, q, prefetch, outwait):
            wait_gather(t, jb)
            if outwait:
                wait_out(t - NOB, q)
            transpose_add(t, jb, q)
            start_out(t, q)
            if prefetch:
                start_gather(t + NBUF, jb)

        for t in range(NOB):
            step(t, t % NBUF, t % NOB, True, False)

        STEP = NBUF * NOB
        body_lo = NOB
        n_mid = ((L - NBUF - body_lo) // STEP) * STEP
        mid_hi = body_lo + n_mid

        @pl.loop(body_lo, mid_hi, step=STEP)
        def _(t0):
            for j in range(STEP):
                step(t0 + j, (body_lo + j) % NBUF, j % NOB, True, True)

        for t in range(mid_hi, L):
            step(t, t % NBUF, t % NOB, t + NBUF < L, True)

        for t in range(L - NOB, L):
            wait_out(t, t % NOB)

    return k2(x_t, tab2, pos_t)


@jax.jit
def kernel(x, token_table, pos_table):
    L = x.shape[1]
    V, E = token_table.shape
    x_t = x.T.astype(jnp.int32)
    tok_t = token_table.T
    pos_t = pos_table.T.astype(jnp.float32)
    vt = (V // CW) * CW
    tail2 = token_table[vt:].reshape((V - vt) // 2, 2 * E)
    tab2 = _sc_pack_table(tok_t, tail2)
    out_t = _sc_embed(x_t, tab2, pos_t)
    return jnp.transpose(out_t, (2, 0, 1))


# speedup vs baseline: 2.8557x; 2.4037x over previous
import dataclasses
import functools

import jax
import jax.numpy as jnp
from jax import lax
from jax.experimental import pallas as pl
from jax.experimental.pallas import tpu as pltpu
from jax.experimental.pallas import tpu_sc as plsc

NC, NS, LANES = 2, 16, 16
NW = NC * NS


def _sc_compiler_params():
    cp = pltpu.CompilerParams(use_tc_tiling_on_sc=False)
    if "needs_layout_passes" in pltpu.CompilerParams.__dataclass_fields__:
        cp = dataclasses.replace(cp, needs_layout_passes=False)
    return cp


def _sc_embed(x_t, tab, pos_et):
    L, B = x_t.shape
    V, E = tab.shape
    BW = B // NW
    ET, BT = E // 8, B // 128
    OWP = 129
    NBUF = 3
    NOB = 2

    mesh = plsc.VectorSubcoreMesh(core_axis_name="c", subcore_axis_name="s")

    @functools.partial(
        pl.kernel,
        out_type=jax.ShapeDtypeStruct((L, ET, BT, 8, 128), jnp.float32),
        mesh=mesh,
        compiler_params=_sc_compiler_params(),
        scratch_types=[
            pltpu.VMEM((L, BW), jnp.int32),
            pltpu.VMEM((NBUF, BW, E), jnp.float32),
            pltpu.VMEM((NOB, ET, 8, OWP), jnp.float32),
            pltpu.VMEM((E, L), jnp.float32),
            pltpu.SemaphoreType.DMA((NBUF,)),
            pltpu.SemaphoreType.DMA((NOB,)),
        ],
    )
    def k(xt_hbm, tab_hbm, pos_hbm, out_hbm, idxs_v, rows_v, outb_v, pos_v,
          gsem, osem):
        wid = lax.axis_index("s") * NC + lax.axis_index("c")
        bbase = wid * BW
        iota = lax.iota(jnp.int32, LANES)

        pltpu.sync_copy(pos_hbm, pos_v)
        pltpu.sync_copy(xt_hbm.at[:, pl.ds(bbase, BW)], idxs_v)

        def start_gather(t, jb):
            pltpu.async_copy(
                tab_hbm.at[idxs_v.at[t]], rows_v.at[jb], gsem.at[jb])

        def wait_gather(t, jb):
            pltpu.make_async_copy(
                tab_hbm.at[idxs_v.at[t]], rows_v.at[jb], gsem.at[jb]).wait()

        def start_out(t, q):
            pltpu.async_copy(
                outb_v.at[q, :, :, pl.ds(0, 128)],
                out_hbm.at[t, :, wid], osem.at[q])

        def wait_out(t, q):
            pltpu.make_async_copy(
                outb_v.at[q, :, :, pl.ds(0, 128)],
                out_hbm.at[t, :, wid], osem.at[q]).wait()

        def transpose_add(t, jb, q):
            lvec = jnp.full((LANES,), t, jnp.int32)
            pos_c = [
                plsc.load_gather(pos_v, [iota + c * LANES, lvec])
                for c in range(E // LANES)
            ]
            e_hi = [(iota + c * LANES) >> 3 for c in range(E // LANES)]
            e_lo = [(iota + c * LANES) & 7 for c in range(E // LANES)]

            @plsc.parallel_loop(0, BW, step=1, unroll=2)
            def _(b):
                bvec = jnp.full((LANES,), b, jnp.int32)
                for c in range(E // LANES):
                    val = rows_v[jb, b, pl.ds(c * LANES, LANES)]
                    plsc.store_scatter(
                        outb_v.at[q], [e_hi[c], e_lo[c], bvec],
                        val + pos_c[c])

        for t in range(NBUF):
            start_gather(t, t)

        def step(t, jb, q, prefetch, outwait):
            wait_gather(t, jb)
            if outwait:
                wait_out(t - NOB, q)
            transpose_add(t, jb, q)
            start_out(t, q)
            if prefetch:
                start_gather(t + NBUF, jb)

        for t in range(NOB):
            step(t, t % NBUF, t % NOB, True, False)

        STEP = NBUF * NOB
        body_lo = NOB
        n_mid = ((L - NBUF - body_lo) // STEP) * STEP
        mid_hi = body_lo + n_mid

        @pl.loop(body_lo, mid_hi, step=STEP)
        def _(t0):
            for j in range(STEP):
                step(t0 + j, (body_lo + j) % NBUF, j % NOB, True, True)

        for t in range(mid_hi, L):
            step(t, t % NBUF, t % NOB, t + NBUF < L, True)

        for t in range(L - NOB, L):
            wait_out(t, t % NOB)

    return k(x_t, tab, pos_et)


@jax.jit
def kernel(x, token_table, pos_table):
    B, L = x.shape
    V, E = token_table.shape
    x_t = x.T.astype(jnp.int32)
    pos_et = pos_table[:L].T.astype(jnp.float32)
    tab2 = lax.optimization_barrier(
        jnp.reshape(token_table, (V // 2, 2 * E)))
    tab = jnp.reshape(tab2, (V, E))
    out5 = _sc_embed(x_t, tab, pos_et)
    out = jnp.transpose(out5, (2, 4, 0, 1, 3))
    return jnp.reshape(out, (B, L, E))
